# Initial kernel scaffold; baseline (speedup 1.0000x reference)
import jax
import jax.numpy as jnp
from jax.experimental import pallas as pl


def kernel(x, edge_index, batch, params):
    raise NotImplementedError("write your pallas kernel here")


# speedup vs baseline: 4.2750x; 4.2750x over previous
import functools

import jax
import jax.numpy as jnp
from jax import lax
from jax.experimental import pallas as pl
from jax.experimental.pallas import tpu as pltpu
from jax.experimental.pallas import tpu_sc as plsc

NC = 2
NS = 16
NW = NC * NS
LANES = 16

EDGE_CHUNK = 80
ZR = 160
ACC_ROWS = 72

NEG_INF = float("-inf")


@functools.lru_cache(maxsize=None)
def _make_sc_scatter(NP, H, E):
    E_tile = E // NW
    n_chunks = E_tile // EDGE_CHUNK
    rows_per_tile = NP // NS
    chunk_plan = []
    off = 0
    while off < rows_per_tile:
        sz = min(ZR, rows_per_tile - off)
        chunk_plan.append((off, sz))
        off += sz

    mesh = plsc.VectorSubcoreMesh(
        core_axis_name="c", subcore_axis_name="s",
        num_cores=NC, num_subcores=NS)

    @functools.partial(
        pl.kernel,
        out_type=jax.ShapeDtypeStruct((NC, NP, H), jnp.float32),
        mesh=mesh,
        scratch_types=[
            pltpu.VMEM((EDGE_CHUNK,), jnp.int32),
            pltpu.VMEM((EDGE_CHUNK,), jnp.int32),
            pltpu.VMEM((EDGE_CHUNK, H), jnp.float32),
            pltpu.VMEM((ZR, H), jnp.float32),
            pltpu.VMEM_SHARED((NP, H), jnp.float32),
            pltpu.SemaphoreType.DMA,
        ],
    )
    def sc_scatter(h_hbm, src_hbm, dst_hbm, zeros_hbm, out_hbm,
                   src_v, dst_v, rows_v, zbuf, acc, sem):
        c = lax.axis_index("c")
        s = lax.axis_index("s")
        row0 = pl.multiple_of(s * rows_per_tile, 8)

        pltpu.sync_copy(zeros_hbm, zbuf)
        for (o, sz) in chunk_plan:
            r = pl.multiple_of(row0 + o, 8)
            pltpu.sync_copy(zbuf.at[pl.ds(0, sz), :],
                            acc.at[pl.ds(r, sz), :])
        plsc.subcore_barrier()

        tile_base = (c * NS + s) * E_tile

        def body(i, carry):
            base = pl.multiple_of(tile_base + i * EDGE_CHUNK, 8)
            pltpu.sync_copy(src_hbm.at[pl.ds(base, EDGE_CHUNK)], src_v)
            pltpu.sync_copy(dst_hbm.at[pl.ds(base, EDGE_CHUNK)], dst_v)
            pltpu.async_copy(h_hbm.at[src_v], rows_v, sem).wait()
            pltpu.sync_copy(rows_v, acc.at[dst_v], add=True)
            return carry

        lax.fori_loop(0, n_chunks, body, 0)
        plsc.subcore_barrier()

        for (o, sz) in chunk_plan:
            r = pl.multiple_of(row0 + o, 8)
            pltpu.sync_copy(acc.at[pl.ds(r, sz), :], zbuf.at[pl.ds(0, sz), :])
            pltpu.sync_copy(zbuf.at[pl.ds(0, sz), :],
                            out_hbm.at[c, pl.ds(r, sz), :])

    return sc_scatter


def _dot_t(a, w):
    return lax.dot_general(a, w, (((1,), (1,)), ((), ())),
                           preferred_element_type=jnp.float32)


@functools.lru_cache(maxsize=None)
def _make_tc_mlp(NP, H, BLK):
    nblk = NP // BLK

    def body(h_ref, p0_ref, p1_ref, w1_ref, b1_ref, w2_ref, b2_ref, o_ref):
        xin = h_ref[...] + p0_ref[0] + p1_ref[0]
        a = jnp.maximum(_dot_t(xin, w1_ref[...]) + b1_ref[...], 0.0)
        o = jnp.maximum(_dot_t(a, w2_ref[...]) + b2_ref[...], 0.0)
        o_ref[...] = o

    return pl.pallas_call(
        body,
        grid=(nblk,),
        in_specs=[
            pl.BlockSpec((BLK, H), lambda b: (b, 0)),
            pl.BlockSpec((1, BLK, H), lambda b: (0, b, 0)),
            pl.BlockSpec((1, BLK, H), lambda b: (1, b, 0)),
            pl.BlockSpec((H, H), lambda b: (0, 0)),
            pl.BlockSpec((1, H), lambda b: (0, 0)),
            pl.BlockSpec((H, H), lambda b: (0, 0)),
            pl.BlockSpec((1, H), lambda b: (0, 0)),
        ],
        out_specs=pl.BlockSpec((BLK, H), lambda b: (b, 0)),
        out_shape=jax.ShapeDtypeStruct((NP, H), jnp.float32),
    )


@functools.lru_cache(maxsize=None)
def _make_tc_mlp_bn(N, NP, H, BLK):
    nblk = NP // BLK

    def body(h_ref, p0_ref, p1_ref, w1_ref, b1_ref, w2_ref, b2_ref,
             g_ref, be_ref, o_ref, h1_s, stats):
        p = pl.program_id(0)
        b = pl.program_id(1)

        @pl.when(p == 0)
        def _phase0():
            xin = h_ref[...] + p0_ref[0] + p1_ref[0]
            h1 = _dot_t(xin, w1_ref[...]) + b1_ref[...]
            h1_s[pl.ds(b * BLK, BLK), :] = h1
            rows = b * BLK + lax.broadcasted_iota(jnp.int32, (BLK, 1), 0)
            valid = rows < N
            h1m = jnp.where(valid, h1, 0.0)
            ssum = jnp.sum(h1m, axis=0, keepdims=True)
            ssq = jnp.sum(h1m * h1m, axis=0, keepdims=True)

            @pl.when(b == 0)
            def _init():
                stats[0:1, :] = ssum
                stats[1:2, :] = ssq

            @pl.when(b != 0)
            def _acc():
                stats[0:1, :] = stats[0:1, :] + ssum
                stats[1:2, :] = stats[1:2, :] + ssq

        @pl.when(p == 1)
        def _phase1():
            mean = stats[0:1, :] / float(N)
            var = stats[1:2, :] / float(N) - mean * mean
            h1 = h1_s[pl.ds(b * BLK, BLK), :]
            hn = (h1 - mean) / jnp.sqrt(var + 1e-5) * g_ref[...] + be_ref[...]
            a = jnp.maximum(hn, 0.0)
            o = jnp.maximum(_dot_t(a, w2_ref[...]) + b2_ref[...], 0.0)
            o_ref[...] = o

    return pl.pallas_call(
        body,
        grid=(2, nblk),
        in_specs=[
            pl.BlockSpec((BLK, H), lambda p, b: (b, 0)),
            pl.BlockSpec((1, BLK, H), lambda p, b: (0, b, 0)),
            pl.BlockSpec((1, BLK, H), lambda p, b: (1, b, 0)),
            pl.BlockSpec((H, H), lambda p, b: (0, 0)),
            pl.BlockSpec((1, H), lambda p, b: (0, 0)),
            pl.BlockSpec((H, H), lambda p, b: (0, 0)),
            pl.BlockSpec((1, H), lambda p, b: (0, 0)),
            pl.BlockSpec((1, H), lambda p, b: (0, 0)),
            pl.BlockSpec((1, H), lambda p, b: (0, 0)),
        ],
        out_specs=pl.BlockSpec((BLK, H), lambda p, b: (b, 0)),
        out_shape=jax.ShapeDtypeStruct((NP, H), jnp.float32),
        scratch_shapes=[
            pltpu.VMEM((NP, H), jnp.float32),
            pltpu.VMEM((8, H), jnp.float32),
        ],
    )


@functools.lru_cache(maxsize=None)
def _make_sc_poolsum(NP, H, NL):
    rpt = NP // NW
    n_chunks = rpt // EDGE_CHUNK

    mesh = plsc.VectorSubcoreMesh(
        core_axis_name="c", subcore_axis_name="s",
        num_cores=NC, num_subcores=NS)

    @functools.partial(
        pl.kernel,
        out_type=jax.ShapeDtypeStruct((NL * NC, ACC_ROWS, H), jnp.float32),
        mesh=mesh,
        scratch_types=[
            pltpu.VMEM((EDGE_CHUNK,), jnp.int32),
            pltpu.VMEM((EDGE_CHUNK, H), jnp.float32),
            pltpu.VMEM_SHARED((ACC_ROWS, H), jnp.float32),
        ],
    )
    def sc_poolsum(*refs):
        hs = refs[:NL]
        batch_hbm = refs[NL]
        zeros_hbm = refs[NL + 1]
        out_hbm = refs[NL + 2]
        idx_v, rows_v, acc = refs[NL + 3:]

        c = lax.axis_index("c")
        s = lax.axis_index("s")
        wid = c * NS + s
        row0 = pl.multiple_of(wid * rpt, 8)

        @pl.when(s == 0)
        def _zero0():
            pltpu.sync_copy(zeros_hbm, acc)
        plsc.subcore_barrier()

        for a in range(NL):
            def body(k, carry):
                base = pl.multiple_of(row0 + k * EDGE_CHUNK, 8)
                pltpu.sync_copy(batch_hbm.at[pl.ds(base, EDGE_CHUNK)], idx_v)
                pltpu.sync_copy(hs[a].at[pl.ds(base, EDGE_CHUNK), :], rows_v)
                pltpu.sync_copy(rows_v, acc.at[idx_v], add=True)
                return carry

            lax.fori_loop(0, n_chunks, body, 0)
            plsc.subcore_barrier()

            @pl.when(s == 0)
            def _drain():
                pltpu.sync_copy(acc, out_hbm.at[a * NC + c])
                if a + 1 < NL:
                    pltpu.sync_copy(zeros_hbm, acc)
            plsc.subcore_barrier()

    return sc_poolsum


@functools.lru_cache(maxsize=None)
def _make_tc_sumreduce(B, H, NL):
    def body(ps_ref, s_ref):
        ps = ps_ref[0].reshape(NC, ACC_ROWS, H)
        s_ref[...] = ps[0, :B] + ps[1, :B]

    return pl.pallas_call(
        body,
        grid=(NL,),
        in_specs=[pl.BlockSpec((1, NC, ACC_ROWS * H), lambda a: (a, 0, 0))],
        out_specs=pl.BlockSpec((B, H), lambda a: (0, a)),
        out_shape=jax.ShapeDtypeStruct((B, NL * H), jnp.float32),
    )


@functools.lru_cache(maxsize=None)
def _make_tc_maxpool(NP, H, B, NL, BLK):
    NB = NP // BLK

    def body(h_ref, b_ref, m_ref, acc):
        blk = pl.program_id(1)

        @pl.when(blk == 0)
        def _init():
            acc[...] = jnp.full((ACC_ROWS, H), NEG_INF, jnp.float32)

        hblk = h_ref[0]
        bv = b_ref[...]
        lo = bv[0, 0]
        hi = bv[BLK - 1, 0]

        def seg_body(seg, carry):
            m = jnp.max(jnp.where(bv == seg, hblk, NEG_INF),
                        axis=0, keepdims=True)
            cur = acc[pl.ds(seg, 1), :]
            acc[pl.ds(seg, 1), :] = jnp.maximum(cur, m)
            return carry

        lax.fori_loop(lo, hi + 1, seg_body, 0)

        @pl.when(blk == NB - 1)
        def _emit():
            m_ref[...] = acc[pl.ds(0, B), :]

    return pl.pallas_call(
        body,
        grid=(NL, NB),
        in_specs=[
            pl.BlockSpec((1, BLK, H), lambda a, b: (a, b, 0)),
            pl.BlockSpec((BLK, 1), lambda a, b: (b, 0)),
        ],
        out_specs=pl.BlockSpec((B, H), lambda a, b: (0, a)),
        out_shape=jax.ShapeDtypeStruct((B, NL * H), jnp.float32),
        scratch_shapes=[pltpu.VMEM((ACC_ROWS, H), jnp.float32)],
    )


@functools.lru_cache(maxsize=None)
def _make_tc_head(B, D, HP):
    def body(x_ref, w1_ref, b1_ref, w2_ref, b2_ref, sig_ref, log_ref):
        t = jnp.maximum(_dot_t(x_ref[...], w1_ref[...]) + b1_ref[...], 0.0)
        lg = _dot_t(t, w2_ref[...]) + b2_ref[...]
        log_ref[...] = lg
        sig_ref[...] = 1.0 / (1.0 + jnp.exp(-lg))

    return pl.pallas_call(
        body,
        out_shape=(jax.ShapeDtypeStruct((B, HP), jnp.float32),
                   jax.ShapeDtypeStruct((B, HP), jnp.float32)),
    )


def kernel(x, edge_index, batch, params):
    N, F = x.shape
    E = edge_index.shape[1]
    H = F
    B = 64
    NL = len(params["convs"])
    NP = ((N + 8 * NW - 1) // (8 * NW)) * (8 * NW)
    BLK = NP // 10

    src = edge_index[0]
    dst = edge_index[1]
    zeros = jnp.zeros((ZR, H), jnp.float32)

    x_pad = jnp.pad(x, ((0, NP - N), (0, 0)))
    batch_pad = jnp.concatenate(
        [batch.astype(jnp.int32), jnp.full((NP - N,), B, jnp.int32)])

    sc_scatter = _make_sc_scatter(NP, H, E)
    tc_mlp = _make_tc_mlp(NP, H, BLK)
    tc_mlp_bn = _make_tc_mlp_bn(N, NP, H, BLK)

    h = x_pad
    hs = []
    for i, layer in enumerate(params["convs"]):
        partials = sc_scatter(h, src, dst, zeros)
        b1 = layer["b1"].reshape(1, H)
        b2 = layer["b2"].reshape(1, H)
        if i == 0:
            g = layer["gamma"].reshape(1, H)
            be = layer["beta"].reshape(1, H)
            h = tc_mlp_bn(h, partials, partials, layer["W1"], b1,
                          layer["W2"], b2, g, be)
        else:
            h = tc_mlp(h, partials, partials, layer["W1"], b1,
                       layer["W2"], b2)
        hs.append(h)

    zeros_acc = jnp.zeros((ACC_ROWS, H), jnp.float32)
    sc_poolsum = _make_sc_poolsum(NP, H, NL)
    psum = sc_poolsum(*hs, batch_pad, zeros_acc)
    sums = _make_tc_sumreduce(B, H, NL)(
        psum.reshape(NL, NC, ACC_ROWS * H))
    maxs = _make_tc_maxpool(NP, H, B, NL, BLK)(
        jnp.stack(hs), batch_pad.reshape(NP, 1))
    pooled = jnp.concatenate([sums, maxs], axis=1)

    D = 2 * NL * H
    HP = 128
    w2p = jnp.zeros((HP, D), jnp.float32).at[0].set(params["lin2_W"][0])
    b2p = jnp.zeros((1, HP), jnp.float32).at[0, 0].set(params["lin2_b"][0])
    head = _make_tc_head(B, D, HP)
    sig, logit = head(pooled, params["lin1_W"],
                      params["lin1_b"].reshape(1, D), w2p, b2p)
    return (sig[:, :1], logit[:, :1])


# speedup vs baseline: 8.0342x; 1.8793x over previous
import functools

import jax
import jax.numpy as jnp
from jax import lax
from jax.experimental import pallas as pl
from jax.experimental.pallas import tpu as pltpu
from jax.experimental.pallas import tpu_sc as plsc

NC = 2
NS = 16
NW = NC * NS
LANES = 16

EDGE_CHUNK = 80
ZR = 80
ACC_ROWS = 72

NEG_INF = float("-inf")


@functools.lru_cache(maxsize=None)
def _make_sc_scatter(NP, H, E):
    E_tile = E // NW
    n_chunks = E_tile // EDGE_CHUNK
    rows_per_tile = NP // NS
    chunk_plan = []
    off = 0
    while off < rows_per_tile:
        sz = min(ZR, rows_per_tile - off)
        chunk_plan.append((off, sz))
        off += sz

    mesh = plsc.VectorSubcoreMesh(
        core_axis_name="c", subcore_axis_name="s",
        num_cores=NC, num_subcores=NS)

    @functools.partial(
        pl.kernel,
        out_type=jax.ShapeDtypeStruct((NC, NP, H), jnp.float32),
        mesh=mesh,
        scratch_types=[
            pltpu.VMEM((E_tile,), jnp.int32),
            pltpu.VMEM((EDGE_CHUNK,), jnp.int32),
            pltpu.VMEM((EDGE_CHUNK,), jnp.int32),
            pltpu.VMEM((EDGE_CHUNK, H), jnp.float32),
            pltpu.VMEM((EDGE_CHUNK, H), jnp.float32),
            pltpu.VMEM((ZR, H), jnp.float32),
            pltpu.VMEM_SHARED((NP, H), jnp.float32),
            pltpu.SemaphoreType.DMA,
            pltpu.SemaphoreType.DMA,
        ],
    )
    def sc_scatter(h_hbm, src_hbm, dst_hbm, zeros_hbm, out_hbm,
                   src_all, dst_v0, dst_v1, rows_v0, rows_v1,
                   zbuf, acc, sem0, sem1):
        c = lax.axis_index("c")
        s = lax.axis_index("s")
        row0 = pl.multiple_of(s * rows_per_tile, 8)
        tile_base = pl.multiple_of((c * NS + s) * E_tile, 8)

        pltpu.sync_copy(src_hbm.at[pl.ds(tile_base, E_tile)], src_all)

        pltpu.sync_copy(zeros_hbm, zbuf)
        for (o, sz) in chunk_plan:
            r = pl.multiple_of(row0 + o, 8)
            pltpu.sync_copy(zbuf.at[pl.ds(0, sz), :],
                            acc.at[pl.ds(r, sz), :])
        plsc.subcore_barrier()

        def src_sl(k):
            return src_all.at[pl.ds(pl.multiple_of(k * EDGE_CHUNK, 8),
                                    EDGE_CHUNK)]

        def start(k, dst_v, rows_v, sem):
            base = pl.multiple_of(tile_base + k * EDGE_CHUNK, 8)
            pltpu.sync_copy(dst_hbm.at[pl.ds(base, EDGE_CHUNK)], dst_v)
            pltpu.async_copy(h_hbm.at[src_sl(k)], rows_v, sem)

        def finish(k, dst_v, rows_v, sem):
            pltpu.make_async_copy(h_hbm.at[src_sl(k)], rows_v, sem).wait()
            pltpu.sync_copy(rows_v, acc.at[dst_v], add=True)

        start(0, dst_v0, rows_v0, sem0)

        def body(i, carry):
            k = i * 2
            start(k + 1, dst_v1, rows_v1, sem1)
            finish(k, dst_v0, rows_v0, sem0)
            start(k + 2, dst_v0, rows_v0, sem0)
            finish(k + 1, dst_v1, rows_v1, sem1)
            return carry

        lax.fori_loop(0, (n_chunks - 1) // 2, body, 0)
        finish(n_chunks - 1, dst_v0, rows_v0, sem0)
        plsc.subcore_barrier()

        for (o, sz) in chunk_plan:
            r = pl.multiple_of(row0 + o, 8)
            pltpu.sync_copy(acc.at[pl.ds(r, sz), :], zbuf.at[pl.ds(0, sz), :])
            pltpu.sync_copy(zbuf.at[pl.ds(0, sz), :],
                            out_hbm.at[c, pl.ds(r, sz), :])

    return sc_scatter


def _dot_t(a, w):
    return lax.dot_general(a, w, (((1,), (1,)), ((), ())),
                           preferred_element_type=jnp.float32)


@functools.lru_cache(maxsize=None)
def _make_tc_mlp(NP, H, BLK):
    nblk = NP // BLK

    def body(h_ref, p0_ref, p1_ref, w1_ref, b1_ref, w2_ref, b2_ref, o_ref):
        xin = h_ref[...] + p0_ref[0] + p1_ref[0]
        a = jnp.maximum(_dot_t(xin, w1_ref[...]) + b1_ref[...], 0.0)
        o = jnp.maximum(_dot_t(a, w2_ref[...]) + b2_ref[...], 0.0)
        o_ref[...] = o

    return pl.pallas_call(
        body,
        grid=(nblk,),
        in_specs=[
            pl.BlockSpec((BLK, H), lambda b: (b, 0)),
            pl.BlockSpec((1, BLK, H), lambda b: (0, b, 0)),
            pl.BlockSpec((1, BLK, H), lambda b: (1, b, 0)),
            pl.BlockSpec((H, H), lambda b: (0, 0)),
            pl.BlockSpec((1, H), lambda b: (0, 0)),
            pl.BlockSpec((H, H), lambda b: (0, 0)),
            pl.BlockSpec((1, H), lambda b: (0, 0)),
        ],
        out_specs=pl.BlockSpec((BLK, H), lambda b: (b, 0)),
        out_shape=jax.ShapeDtypeStruct((NP, H), jnp.float32),
    )


@functools.lru_cache(maxsize=None)
def _make_tc_mlp_bn(N, NP, H, BLK):
    nblk = NP // BLK

    def body(h_ref, p0_ref, p1_ref, w1_ref, b1_ref, w2_ref, b2_ref,
             g_ref, be_ref, o_ref, h1_s, stats):
        p = pl.program_id(0)
        b = pl.program_id(1)

        @pl.when(p == 0)
        def _phase0():
            xin = h_ref[...] + p0_ref[0] + p1_ref[0]
            h1 = _dot_t(xin, w1_ref[...]) + b1_ref[...]
            h1_s[pl.ds(b * BLK, BLK), :] = h1
            rows = b * BLK + lax.broadcasted_iota(jnp.int32, (BLK, 1), 0)
            valid = rows < N
            h1m = jnp.where(valid, h1, 0.0)
            ssum = jnp.sum(h1m, axis=0, keepdims=True)
            ssq = jnp.sum(h1m * h1m, axis=0, keepdims=True)

            @pl.when(b == 0)
            def _init():
                stats[0:1, :] = ssum
                stats[1:2, :] = ssq

            @pl.when(b != 0)
            def _acc():
                stats[0:1, :] = stats[0:1, :] + ssum
                stats[1:2, :] = stats[1:2, :] + ssq

        @pl.when(p == 1)
        def _phase1():
            mean = stats[0:1, :] / float(N)
            var = stats[1:2, :] / float(N) - mean * mean
            h1 = h1_s[pl.ds(b * BLK, BLK), :]
            hn = (h1 - mean) / jnp.sqrt(var + 1e-5) * g_ref[...] + be_ref[...]
            a = jnp.maximum(hn, 0.0)
            o = jnp.maximum(_dot_t(a, w2_ref[...]) + b2_ref[...], 0.0)
            o_ref[...] = o

    return pl.pallas_call(
        body,
        grid=(2, nblk),
        in_specs=[
            pl.BlockSpec((BLK, H), lambda p, b: (b, 0)),
            pl.BlockSpec((1, BLK, H), lambda p, b: (0, b, 0)),
            pl.BlockSpec((1, BLK, H), lambda p, b: (1, b, 0)),
            pl.BlockSpec((H, H), lambda p, b: (0, 0)),
            pl.BlockSpec((1, H), lambda p, b: (0, 0)),
            pl.BlockSpec((H, H), lambda p, b: (0, 0)),
            pl.BlockSpec((1, H), lambda p, b: (0, 0)),
            pl.BlockSpec((1, H), lambda p, b: (0, 0)),
            pl.BlockSpec((1, H), lambda p, b: (0, 0)),
        ],
        out_specs=pl.BlockSpec((BLK, H), lambda p, b: (b, 0)),
        out_shape=jax.ShapeDtypeStruct((NP, H), jnp.float32),
        scratch_shapes=[
            pltpu.VMEM((NP, H), jnp.float32),
            pltpu.VMEM((8, H), jnp.float32),
        ],
    )


@functools.lru_cache(maxsize=None)
def _make_sc_poolsum(NP, H, NL):
    rpt = NP // NW
    n_chunks = rpt // EDGE_CHUNK

    mesh = plsc.VectorSubcoreMesh(
        core_axis_name="c", subcore_axis_name="s",
        num_cores=NC, num_subcores=NS)

    @functools.partial(
        pl.kernel,
        out_type=jax.ShapeDtypeStruct((NL * NC, ACC_ROWS, H), jnp.float32),
        mesh=mesh,
        scratch_types=[
            pltpu.VMEM((EDGE_CHUNK,), jnp.int32),
            pltpu.VMEM((EDGE_CHUNK, H), jnp.float32),
            pltpu.VMEM_SHARED((ACC_ROWS, H), jnp.float32),
        ],
    )
    def sc_poolsum(*refs):
        hs = refs[:NL]
        batch_hbm = refs[NL]
        zeros_hbm = refs[NL + 1]
        out_hbm = refs[NL + 2]
        idx_v, rows_v, acc = refs[NL + 3:]

        c = lax.axis_index("c")
        s = lax.axis_index("s")
        wid = c * NS + s
        row0 = pl.multiple_of(wid * rpt, 8)

        @pl.when(s == 0)
        def _zero0():
            pltpu.sync_copy(zeros_hbm, acc)
        plsc.subcore_barrier()

        for a in range(NL):
            def body(k, carry):
                base = pl.multiple_of(row0 + k * EDGE_CHUNK, 8)
                pltpu.sync_copy(batch_hbm.at[pl.ds(base, EDGE_CHUNK)], idx_v)
                pltpu.sync_copy(hs[a].at[pl.ds(base, EDGE_CHUNK), :], rows_v)
                pltpu.sync_copy(rows_v, acc.at[idx_v], add=True)
                return carry

            lax.fori_loop(0, n_chunks, body, 0)
            plsc.subcore_barrier()

            @pl.when(s == 0)
            def _drain():
                pltpu.sync_copy(acc, out_hbm.at[a * NC + c])
                if a + 1 < NL:
                    pltpu.sync_copy(zeros_hbm, acc)
            plsc.subcore_barrier()

    return sc_poolsum


@functools.lru_cache(maxsize=None)
def _make_tc_sumreduce(B, H, NL):
    def body(ps_ref, s_ref):
        ps = ps_ref[0].reshape(NC, ACC_ROWS, H)
        s_ref[...] = ps[0, :B] + ps[1, :B]

    return pl.pallas_call(
        body,
        grid=(NL,),
        in_specs=[pl.BlockSpec((1, NC, ACC_ROWS * H), lambda a: (a, 0, 0))],
        out_specs=pl.BlockSpec((B, H), lambda a: (0, a)),
        out_shape=jax.ShapeDtypeStruct((B, NL * H), jnp.float32),
    )


@functools.lru_cache(maxsize=None)
def _make_tc_maxpool(NP, H, B, NL, BLK):
    NB = NP // BLK

    def body(h_ref, b_ref, m_ref, acc):
        blk = pl.program_id(1)

        @pl.when(blk == 0)
        def _init():
            acc[...] = jnp.full((ACC_ROWS, H), NEG_INF, jnp.float32)

        hblk = h_ref[0]
        bv = b_ref[...]
        lo = bv[0, 0]
        hi = bv[BLK - 1, 0]

        def seg_body(seg, carry):
            m = jnp.max(jnp.where(bv == seg, hblk, NEG_INF),
                        axis=0, keepdims=True)
            cur = acc[pl.ds(seg, 1), :]
            acc[pl.ds(seg, 1), :] = jnp.maximum(cur, m)
            return carry

        lax.fori_loop(lo, hi + 1, seg_body, 0)

        @pl.when(blk == NB - 1)
        def _emit():
            m_ref[...] = acc[pl.ds(0, B), :]

    return pl.pallas_call(
        body,
        grid=(NL, NB),
        in_specs=[
            pl.BlockSpec((1, BLK, H), lambda a, b: (a, b, 0)),
            pl.BlockSpec((BLK, 1), lambda a, b: (b, 0)),
        ],
        out_specs=pl.BlockSpec((B, H), lambda a, b: (0, a)),
        out_shape=jax.ShapeDtypeStruct((B, NL * H), jnp.float32),
        scratch_shapes=[pltpu.VMEM((ACC_ROWS, H), jnp.float32)],
    )


@functools.lru_cache(maxsize=None)
def _make_tc_head(B, D, HP):
    def body(x_ref, w1_ref, b1_ref, w2_ref, b2_ref, sig_ref, log_ref):
        t = jnp.maximum(_dot_t(x_ref[...], w1_ref[...]) + b1_ref[...], 0.0)
        lg = _dot_t(t, w2_ref[...]) + b2_ref[...]
        log_ref[...] = lg
        sig_ref[...] = 1.0 / (1.0 + jnp.exp(-lg))

    return pl.pallas_call(
        body,
        out_shape=(jax.ShapeDtypeStruct((B, HP), jnp.float32),
                   jax.ShapeDtypeStruct((B, HP), jnp.float32)),
    )


def kernel(x, edge_index, batch, params):
    N, F = x.shape
    E = edge_index.shape[1]
    H = F
    B = 64
    NL = len(params["convs"])
    NP = ((N + 8 * NW - 1) // (8 * NW)) * (8 * NW)
    BLK = NP // 10

    src = edge_index[0]
    dst = edge_index[1]
    zeros = jnp.zeros((ZR, H), jnp.float32)

    x_pad = jnp.pad(x, ((0, NP - N), (0, 0)))
    batch_pad = jnp.concatenate(
        [batch.astype(jnp.int32), jnp.full((NP - N,), B, jnp.int32)])

    sc_scatter = _make_sc_scatter(NP, H, E)
    tc_mlp = _make_tc_mlp(NP, H, BLK)
    tc_mlp_bn = _make_tc_mlp_bn(N, NP, H, BLK)

    h = x_pad
    hs = []
    for i, layer in enumerate(params["convs"]):
        partials = sc_scatter(h, src, dst, zeros)
        b1 = layer["b1"].reshape(1, H)
        b2 = layer["b2"].reshape(1, H)
        if i == 0:
            g = layer["gamma"].reshape(1, H)
            be = layer["beta"].reshape(1, H)
            h = tc_mlp_bn(h, partials, partials, layer["W1"], b1,
                          layer["W2"], b2, g, be)
        else:
            h = tc_mlp(h, partials, partials, layer["W1"], b1,
                       layer["W2"], b2)
        hs.append(h)

    zeros_acc = jnp.zeros((ACC_ROWS, H), jnp.float32)
    sc_poolsum = _make_sc_poolsum(NP, H, NL)
    psum = sc_poolsum(*hs, batch_pad, zeros_acc)
    sums = _make_tc_sumreduce(B, H, NL)(
        psum.reshape(NL, NC, ACC_ROWS * H))
    maxs = _make_tc_maxpool(NP, H, B, NL, BLK)(
        jnp.stack(hs), batch_pad.reshape(NP, 1))
    pooled = jnp.concatenate([sums, maxs], axis=1)

    D = 2 * NL * H
    HP = 128
    w2p = jnp.zeros((HP, D), jnp.float32).at[0].set(params["lin2_W"][0])
    b2p = jnp.zeros((1, HP), jnp.float32).at[0, 0].set(params["lin2_b"][0])
    head = _make_tc_head(B, D, HP)
    sig, logit = head(pooled, params["lin1_W"],
                      params["lin1_b"].reshape(1, D), w2p, b2p)
    return (sig[:, :1], logit[:, :1])


# speedup vs baseline: 9.0834x; 1.1306x over previous
import functools

import jax
import jax.numpy as jnp
from jax import lax
from jax.experimental import pallas as pl
from jax.experimental.pallas import tpu as pltpu
from jax.experimental.pallas import tpu_sc as plsc

NC = 2
NS = 16
NW = NC * NS
LANES = 16

EDGE_CHUNK = 80
ZR = 80
ACC_ROWS = 72

NEG_INF = float("-inf")


@functools.lru_cache(maxsize=None)
def _make_sc_scatter(NP, H, E):
    E_tile = E // NW
    CH = 128
    n_full = E_tile // CH
    tail = E_tile - n_full * CH
    rows_per_tile = NP // NS

    mesh = plsc.VectorSubcoreMesh(
        core_axis_name="c", subcore_axis_name="s",
        num_cores=NC, num_subcores=NS)

    @functools.partial(
        pl.kernel,
        out_type=jax.ShapeDtypeStruct((NC, NP, H), jnp.float32),
        mesh=mesh,
        scratch_types=[
            pltpu.VMEM((E_tile,), jnp.int32),
            pltpu.VMEM((CH,), jnp.int32),
            pltpu.VMEM((CH,), jnp.int32),
            pltpu.VMEM((tail,), jnp.int32),
            pltpu.VMEM((CH, H), jnp.float32),
            pltpu.VMEM((CH, H), jnp.float32),
            pltpu.VMEM_SHARED((NP, H), jnp.float32),
            pltpu.SemaphoreType.DMA,
            pltpu.SemaphoreType.DMA,
        ],
    )
    def sc_scatter(h_hbm, src_hbm, dst_hbm, zeros_hbm, out_hbm,
                   src_all, dst_v0, dst_v1, dst_vt, rows_v0, rows_v1,
                   acc, sem0, sem1):
        c = lax.axis_index("c")
        s = lax.axis_index("s")
        row0 = pl.multiple_of(s * rows_per_tile, 8)
        tile_base = pl.multiple_of((c * NS + s) * E_tile, 8)

        pltpu.sync_copy(src_hbm.at[pl.ds(tile_base, E_tile)], src_all)

        pltpu.sync_copy(zeros_hbm,
                        acc.at[pl.ds(row0, rows_per_tile), :])
        plsc.subcore_barrier()

        def src_sl(k, n=CH):
            return src_all.at[pl.ds(pl.multiple_of(k * CH, 8), n)]

        def start(k, dst_v, rows_v, sem):
            base = pl.multiple_of(tile_base + k * CH, 8)
            pltpu.sync_copy(dst_hbm.at[pl.ds(base, CH)], dst_v)
            pltpu.async_copy(h_hbm.at[src_sl(k)], rows_v, sem)

        def finish(k, dst_v, rows_v, sem):
            pltpu.make_async_copy(h_hbm.at[src_sl(k)], rows_v, sem).wait()
            pltpu.sync_copy(rows_v, acc.at[dst_v], add=True)

        start(0, dst_v0, rows_v0, sem0)
        start(1, dst_v1, rows_v1, sem1)

        def body(i, carry):
            k = i * 2
            finish(k, dst_v0, rows_v0, sem0)
            start(k + 2, dst_v0, rows_v0, sem0)
            finish(k + 1, dst_v1, rows_v1, sem1)
            start(k + 3, dst_v1, rows_v1, sem1)
            return carry

        lax.fori_loop(0, (n_full - 2) // 2, body, 0)
        finish(n_full - 2, dst_v0, rows_v0, sem0)
        finish(n_full - 1, dst_v1, rows_v1, sem1)

        if tail:
            tbase = pl.multiple_of(tile_base + n_full * CH, 8)
            pltpu.sync_copy(dst_hbm.at[pl.ds(tbase, tail)], dst_vt)
            pltpu.async_copy(h_hbm.at[src_sl(n_full, tail)],
                             rows_v0.at[pl.ds(0, tail), :], sem0).wait()
            pltpu.sync_copy(rows_v0.at[pl.ds(0, tail), :],
                            acc.at[dst_vt], add=True)
        plsc.subcore_barrier()

        pltpu.sync_copy(acc.at[pl.ds(row0, rows_per_tile), :],
                        out_hbm.at[c, pl.ds(row0, rows_per_tile), :])

    return sc_scatter


def _dot_t(a, w):
    return lax.dot_general(a, w, (((1,), (1,)), ((), ())),
                           preferred_element_type=jnp.float32)


@functools.lru_cache(maxsize=None)
def _make_tc_mlp(NP, H, BLK):
    nblk = NP // BLK

    def body(h_ref, p0_ref, p1_ref, w1_ref, b1_ref, w2_ref, b2_ref, o_ref):
        xin = h_ref[...] + p0_ref[0] + p1_ref[0]
        a = jnp.maximum(_dot_t(xin, w1_ref[...]) + b1_ref[...], 0.0)
        o = jnp.maximum(_dot_t(a, w2_ref[...]) + b2_ref[...], 0.0)
        o_ref[...] = o

    return pl.pallas_call(
        body,
        grid=(nblk,),
        in_specs=[
            pl.BlockSpec((BLK, H), lambda b: (b, 0)),
            pl.BlockSpec((1, BLK, H), lambda b: (0, b, 0)),
            pl.BlockSpec((1, BLK, H), lambda b: (1, b, 0)),
            pl.BlockSpec((H, H), lambda b: (0, 0)),
            pl.BlockSpec((1, H), lambda b: (0, 0)),
            pl.BlockSpec((H, H), lambda b: (0, 0)),
            pl.BlockSpec((1, H), lambda b: (0, 0)),
        ],
        out_specs=pl.BlockSpec((BLK, H), lambda b: (b, 0)),
        out_shape=jax.ShapeDtypeStruct((NP, H), jnp.float32),
    )


@functools.lru_cache(maxsize=None)
def _make_tc_mlp_bn(N, NP, H, BLK):
    nblk = NP // BLK

    def body(h_ref, p0_ref, p1_ref, w1_ref, b1_ref, w2_ref, b2_ref,
             g_ref, be_ref, o_ref, h1_s, stats):
        p = pl.program_id(0)
        b = pl.program_id(1)

        @pl.when(p == 0)
        def _phase0():
            xin = h_ref[...] + p0_ref[0] + p1_ref[0]
            h1 = _dot_t(xin, w1_ref[...]) + b1_ref[...]
            h1_s[pl.ds(b * BLK, BLK), :] = h1
            rows = b * BLK + lax.broadcasted_iota(jnp.int32, (BLK, 1), 0)
            valid = rows < N
            h1m = jnp.where(valid, h1, 0.0)
            ssum = jnp.sum(h1m, axis=0, keepdims=True)
            ssq = jnp.sum(h1m * h1m, axis=0, keepdims=True)

            @pl.when(b == 0)
            def _init():
                stats[0:1, :] = ssum
                stats[1:2, :] = ssq

            @pl.when(b != 0)
            def _acc():
                stats[0:1, :] = stats[0:1, :] + ssum
                stats[1:2, :] = stats[1:2, :] + ssq

        @pl.when(p == 1)
        def _phase1():
            mean = stats[0:1, :] / float(N)
            var = stats[1:2, :] / float(N) - mean * mean
            h1 = h1_s[pl.ds(b * BLK, BLK), :]
            hn = (h1 - mean) / jnp.sqrt(var + 1e-5) * g_ref[...] + be_ref[...]
            a = jnp.maximum(hn, 0.0)
            o = jnp.maximum(_dot_t(a, w2_ref[...]) + b2_ref[...], 0.0)
            o_ref[...] = o

    return pl.pallas_call(
        body,
        grid=(2, nblk),
        in_specs=[
            pl.BlockSpec((BLK, H), lambda p, b: (b, 0)),
            pl.BlockSpec((1, BLK, H), lambda p, b: (0, b, 0)),
            pl.BlockSpec((1, BLK, H), lambda p, b: (1, b, 0)),
            pl.BlockSpec((H, H), lambda p, b: (0, 0)),
            pl.BlockSpec((1, H), lambda p, b: (0, 0)),
            pl.BlockSpec((H, H), lambda p, b: (0, 0)),
            pl.BlockSpec((1, H), lambda p, b: (0, 0)),
            pl.BlockSpec((1, H), lambda p, b: (0, 0)),
            pl.BlockSpec((1, H), lambda p, b: (0, 0)),
        ],
        out_specs=pl.BlockSpec((BLK, H), lambda p, b: (b, 0)),
        out_shape=jax.ShapeDtypeStruct((NP, H), jnp.float32),
        scratch_shapes=[
            pltpu.VMEM((NP, H), jnp.float32),
            pltpu.VMEM((8, H), jnp.float32),
        ],
    )


@functools.lru_cache(maxsize=None)
def _make_sc_poolsum(NP, H, NL):
    rpt = NP // NW
    n_chunks = rpt // EDGE_CHUNK

    mesh = plsc.VectorSubcoreMesh(
        core_axis_name="c", subcore_axis_name="s",
        num_cores=NC, num_subcores=NS)

    @functools.partial(
        pl.kernel,
        out_type=jax.ShapeDtypeStruct((NL * NC, ACC_ROWS, H), jnp.float32),
        mesh=mesh,
        scratch_types=[
            pltpu.VMEM((EDGE_CHUNK,), jnp.int32),
            pltpu.VMEM((EDGE_CHUNK, H), jnp.float32),
            pltpu.VMEM_SHARED((ACC_ROWS, H), jnp.float32),
        ],
    )
    def sc_poolsum(*refs):
        hs = refs[:NL]
        batch_hbm = refs[NL]
        zeros_hbm = refs[NL + 1]
        out_hbm = refs[NL + 2]
        idx_v, rows_v, acc = refs[NL + 3:]

        c = lax.axis_index("c")
        s = lax.axis_index("s")
        wid = c * NS + s
        row0 = pl.multiple_of(wid * rpt, 8)

        @pl.when(s == 0)
        def _zero0():
            pltpu.sync_copy(zeros_hbm, acc)
        plsc.subcore_barrier()

        for a in range(NL):
            def body(k, carry):
                base = pl.multiple_of(row0 + k * EDGE_CHUNK, 8)
                pltpu.sync_copy(batch_hbm.at[pl.ds(base, EDGE_CHUNK)], idx_v)
                pltpu.sync_copy(hs[a].at[pl.ds(base, EDGE_CHUNK), :], rows_v)
                pltpu.sync_copy(rows_v, acc.at[idx_v], add=True)
                return carry

            lax.fori_loop(0, n_chunks, body, 0)
            plsc.subcore_barrier()

            @pl.when(s == 0)
            def _drain():
                pltpu.sync_copy(acc, out_hbm.at[a * NC + c])
                if a + 1 < NL:
                    pltpu.sync_copy(zeros_hbm, acc)
            plsc.subcore_barrier()

    return sc_poolsum


@functools.lru_cache(maxsize=None)
def _make_tc_sumreduce(B, H, NL):
    def body(ps_ref, s_ref):
        ps = ps_ref[0].reshape(NC, ACC_ROWS, H)
        s_ref[...] = ps[0, :B] + ps[1, :B]

    return pl.pallas_call(
        body,
        grid=(NL,),
        in_specs=[pl.BlockSpec((1, NC, ACC_ROWS * H), lambda a: (a, 0, 0))],
        out_specs=pl.BlockSpec((B, H), lambda a: (0, a)),
        out_shape=jax.ShapeDtypeStruct((B, NL * H), jnp.float32),
    )


@functools.lru_cache(maxsize=None)
def _make_tc_maxpool(NP, H, B, NL, BLK):
    NB = NP // BLK

    def body(h_ref, b_ref, m_ref, acc):
        blk = pl.program_id(1)

        @pl.when(blk == 0)
        def _init():
            acc[...] = jnp.full((ACC_ROWS, H), NEG_INF, jnp.float32)

        hblk = h_ref[0]
        bv = b_ref[...]
        lo = bv[0, 0]
        hi = bv[BLK - 1, 0]

        def seg_body(seg, carry):
            m = jnp.max(jnp.where(bv == seg, hblk, NEG_INF),
                        axis=0, keepdims=True)
            cur = acc[pl.ds(seg, 1), :]
            acc[pl.ds(seg, 1), :] = jnp.maximum(cur, m)
            return carry

        lax.fori_loop(lo, hi + 1, seg_body, 0)

        @pl.when(blk == NB - 1)
        def _emit():
            m_ref[...] = acc[pl.ds(0, B), :]

    return pl.pallas_call(
        body,
        grid=(NL, NB),
        in_specs=[
            pl.BlockSpec((1, BLK, H), lambda a, b: (a, b, 0)),
            pl.BlockSpec((BLK, 1), lambda a, b: (b, 0)),
        ],
        out_specs=pl.BlockSpec((B, H), lambda a, b: (0, a)),
        out_shape=jax.ShapeDtypeStruct((B, NL * H), jnp.float32),
        scratch_shapes=[pltpu.VMEM((ACC_ROWS, H), jnp.float32)],
    )


@functools.lru_cache(maxsize=None)
def _make_tc_head(B, D, HP):
    def body(x_ref, w1_ref, b1_ref, w2_ref, b2_ref, sig_ref, log_ref):
        t = jnp.maximum(_dot_t(x_ref[...], w1_ref[...]) + b1_ref[...], 0.0)
        lg = _dot_t(t, w2_ref[...]) + b2_ref[...]
        log_ref[...] = lg
        sig_ref[...] = 1.0 / (1.0 + jnp.exp(-lg))

    return pl.pallas_call(
        body,
        out_shape=(jax.ShapeDtypeStruct((B, HP), jnp.float32),
                   jax.ShapeDtypeStruct((B, HP), jnp.float32)),
    )


def kernel(x, edge_index, batch, params):
    N, F = x.shape
    E = edge_index.shape[1]
    H = F
    B = 64
    NL = len(params["convs"])
    NP = ((N + 8 * NW - 1) // (8 * NW)) * (8 * NW)
    BLK = NP // 10

    src = edge_index[0]
    dst = edge_index[1]
    zeros = jnp.zeros((NP // NS, H), jnp.float32)

    x_pad = jnp.pad(x, ((0, NP - N), (0, 0)))
    batch_pad = jnp.concatenate(
        [batch.astype(jnp.int32), jnp.full((NP - N,), B, jnp.int32)])

    sc_scatter = _make_sc_scatter(NP, H, E)
    tc_mlp = _make_tc_mlp(NP, H, BLK)
    tc_mlp_bn = _make_tc_mlp_bn(N, NP, H, BLK)

    h = x_pad
    hs = []
    for i, layer in enumerate(params["convs"]):
        partials = sc_scatter(h, src, dst, zeros)
        b1 = layer["b1"].reshape(1, H)
        b2 = layer["b2"].reshape(1, H)
        if i == 0:
            g = layer["gamma"].reshape(1, H)
            be = layer["beta"].reshape(1, H)
            h = tc_mlp_bn(h, partials, partials, layer["W1"], b1,
                          layer["W2"], b2, g, be)
        else:
            h = tc_mlp(h, partials, partials, layer["W1"], b1,
                       layer["W2"], b2)
        hs.append(h)

    zeros_acc = jnp.zeros((ACC_ROWS, H), jnp.float32)
    sc_poolsum = _make_sc_poolsum(NP, H, NL)
    psum = sc_poolsum(*hs, batch_pad, zeros_acc)
    sums = _make_tc_sumreduce(B, H, NL)(
        psum.reshape(NL, NC, ACC_ROWS * H))
    maxs = _make_tc_maxpool(NP, H, B, NL, BLK)(
        jnp.stack(hs), batch_pad.reshape(NP, 1))
    pooled = jnp.concatenate([sums, maxs], axis=1)

    D = 2 * NL * H
    HP = 128
    w2p = jnp.zeros((HP, D), jnp.float32).at[0].set(params["lin2_W"][0])
    b2p = jnp.zeros((1, HP), jnp.float32).at[0, 0].set(params["lin2_b"][0])
    head = _make_tc_head(B, D, HP)
    sig, logit = head(pooled, params["lin1_W"],
                      params["lin1_b"].reshape(1, D), w2p, b2p)
    return (sig[:, :1], logit[:, :1])
